# Initial kernel scaffold; baseline (speedup 1.0000x reference)
import jax
import jax.numpy as jnp
from jax.experimental import pallas as pl


def kernel(x, indices):
    raise NotImplementedError("write your pallas kernel here")


# speedup vs baseline: 1.2297x; 1.2297x over previous
import functools

import jax
import jax.numpy as jnp
from jax import lax
from jax.experimental import pallas as pl
from jax.experimental.pallas import tpu as pltpu
from jax.experimental.pallas import tpu_sc as plsc

B, P, C = 2, 12000, 64
NXY = 512 * 512
NQ = 4
CHUNK = NXY // NQ
LANES = 16
NC, NS = 2, 16
C_PER_TEC = C // NS


def _sc_body(flat_hbm, xt_hbm, out_hbm, idx_buf, x_buf, chunk_buf):
    wid = lax.axis_index("s") * NC + lax.axis_index("c")
    b = wid // NS
    c0 = (wid % NS) * C_PER_TEC

    pltpu.sync_copy(flat_hbm.at[b], idx_buf)

    for ci in range(C_PER_TEC):
        c = c0 + ci
        pltpu.sync_copy(xt_hbm.at[b * C + c], x_buf)
        for q in range(NQ):
            base = q * CHUNK

            def zero_body(i, carry):
                chunk_buf[pl.ds(i * LANES, LANES)] = jnp.zeros(
                    (LANES,), jnp.float32)
                return carry

            lax.fori_loop(0, CHUNK // LANES, zero_body, 0)

            def scatter_body(i, carry):
                idx16 = idx_buf[pl.ds(i * LANES, LANES)]
                off = idx16 - base
                m = (off >= 0) & (off < CHUNK)
                off = jnp.where(m, off, 0)
                xv = x_buf[pl.ds(i * LANES, LANES)]
                plsc.addupdate_scatter(chunk_buf, [off], xv, mask=m)
                return carry

            lax.fori_loop(0, P // LANES, scatter_body, 0)

            row = (b * C + c) * NQ + q
            pltpu.sync_copy(chunk_buf, out_hbm.at[row])


@functools.partial(
    pl.kernel,
    out_type=jax.ShapeDtypeStruct((B * C * NQ, CHUNK), jnp.float32),
    mesh=plsc.VectorSubcoreMesh(
        core_axis_name="c", subcore_axis_name="s",
        num_cores=NC, num_subcores=NS),
    scratch_types=[
        pltpu.VMEM((P,), jnp.int32),
        pltpu.VMEM((P,), jnp.float32),
        pltpu.VMEM((CHUNK,), jnp.float32),
    ],
    compiler_params=pltpu.CompilerParams(needs_layout_passes=False),
)
def _scatter_planes(flat_hbm, xt_hbm, out_hbm, idx_buf, x_buf, chunk_buf):
    _sc_body(flat_hbm, xt_hbm, out_hbm, idx_buf, x_buf, chunk_buf)


def kernel(x, indices):
    flat = indices[:, :, 0] * 512 + indices[:, :, 1]
    xt = jnp.transpose(x, (0, 2, 1)).reshape(B * C, P)
    out = _scatter_planes(flat, xt)
    return out.reshape(B, C, 512, 512)


# speedup vs baseline: 2.2664x; 1.8431x over previous
import functools

import jax
import jax.numpy as jnp
from jax import lax
from jax.experimental import pallas as pl
from jax.experimental.pallas import tpu as pltpu
from jax.experimental.pallas import tpu_sc as plsc

B, P, C = 2, 12000, 64
NXY = 512 * 512
NQ = 4
CHUNK = NXY // NQ
LANES = 16
NC, NS = 2, 16
C_PER_TEC = C // NS


def _sc_body(flat_hbm, xt_hbm, out_hbm, idx_buf, x_buf, chunk_buf):
    wid = lax.axis_index("s") * NC + lax.axis_index("c")
    b = wid // NS
    c0 = (wid % NS) * C_PER_TEC

    pltpu.sync_copy(flat_hbm.at[b], idx_buf)

    for ci in range(C_PER_TEC):
        c = c0 + ci
        pltpu.sync_copy(xt_hbm.at[b * C + c], x_buf)
        for q in range(NQ):
            base = q * CHUNK

            zeros16 = jnp.zeros((LANES,), jnp.float32)
            ZU = 16

            def zero_body(i, carry):
                for k in range(ZU):
                    chunk_buf[pl.ds(i * (LANES * ZU) + k * LANES, LANES)] = (
                        zeros16)
                return carry

            lax.fori_loop(0, CHUNK // (LANES * ZU), zero_body, 0)

            SU = 5

            def scatter_body(i, carry):
                for k in range(SU):
                    sl = pl.ds((i * SU + k) * LANES, LANES)
                    off = idx_buf[sl] - base
                    m = (off >= 0) & (off < CHUNK)
                    off = jnp.where(m, off, 0)
                    plsc.addupdate_scatter(chunk_buf, [off], x_buf[sl],
                                           mask=m)
                return carry

            lax.fori_loop(0, P // (LANES * SU), scatter_body, 0)

            row = (b * C + c) * NQ + q
            pltpu.sync_copy(chunk_buf, out_hbm.at[row])


@functools.partial(
    pl.kernel,
    out_type=jax.ShapeDtypeStruct((B * C * NQ, CHUNK), jnp.float32),
    mesh=plsc.VectorSubcoreMesh(
        core_axis_name="c", subcore_axis_name="s",
        num_cores=NC, num_subcores=NS),
    scratch_types=[
        pltpu.VMEM((P,), jnp.int32),
        pltpu.VMEM((P,), jnp.float32),
        pltpu.VMEM((CHUNK,), jnp.float32),
    ],
    compiler_params=pltpu.CompilerParams(needs_layout_passes=False),
)
def _scatter_planes(flat_hbm, xt_hbm, out_hbm, idx_buf, x_buf, chunk_buf):
    _sc_body(flat_hbm, xt_hbm, out_hbm, idx_buf, x_buf, chunk_buf)


def kernel(x, indices):
    flat = indices[:, :, 0] * 512 + indices[:, :, 1]
    xt = jnp.transpose(x, (0, 2, 1)).reshape(B * C, P)
    out = _scatter_planes(flat, xt)
    return out.reshape(B, C, 512, 512)
